# Initial kernel scaffold; baseline (speedup 1.0000x reference)
import jax
import jax.numpy as jnp
from jax.experimental import pallas as pl


def kernel(x, edge_index, edge_weight, W1, b1, W2, b2):
    raise NotImplementedError("write your pallas kernel here")


# speedup vs baseline: 27.6189x; 27.6189x over previous
import functools

import jax
import jax.numpy as jnp
from jax import lax
from jax.experimental import pallas as pl
from jax.experimental.pallas import tpu as pltpu
from jax.experimental.pallas import tpu_sc as plsc

N = 10000
E = 320000
D = 128
H = 16
NC, NS = 2, 16
NW = NC * NS
EPT = E // NW
CH = 80
NCHK = EPT // CH
NPAD = 10240
DPT = NPAD // NS
RPT = NPAD // NS

_MESH = plsc.VectorSubcoreMesh(core_axis_name="c", subcore_axis_name="s")

_GDN = lax.GatherDimensionNumbers(
    offset_dims=(), collapsed_slice_dims=(0,), start_index_map=(0,))


def _bcast(v, k):
    return lax.gather(v, jnp.full((16, 1), k, jnp.int32), _GDN, (1,),
                      mode=lax.GatherScatterMode.PROMISE_IN_BOUNDS)


@functools.partial(
    pl.kernel,
    out_type=jax.ShapeDtypeStruct((NC, NPAD), jnp.float32),
    mesh=_MESH,
    compiler_params=pltpu.CompilerParams(use_tc_tiling_on_sc=False),
    scratch_types=[
        pltpu.VMEM((NCHK, CH), jnp.int32),
        pltpu.VMEM((NCHK, CH), jnp.float32),
        pltpu.VMEM((DPT,), jnp.float32),
        pltpu.VMEM_SHARED((NPAD,), jnp.float32),
    ],
)
def _deg_kernel(col_hbm, ew_hbm, out_hbm, col_v, w_v, z_v, deg_sh):
    c = lax.axis_index("c")
    s = lax.axis_index("s")
    wid = c * NS + s

    @pl.loop(0, DPT // 16)
    def _(i):
        z_v[pl.ds(i * 16, 16)] = jnp.zeros((16,), jnp.float32)

    pltpu.sync_copy(z_v, deg_sh.at[pl.ds(s * DPT, DPT)])
    pltpu.sync_copy(col_hbm.at[wid], col_v)
    pltpu.sync_copy(ew_hbm.at[wid], w_v)
    plsc.subcore_barrier()

    @pl.loop(0, NCHK)
    def _(j):
        pltpu.sync_copy(w_v.at[j], deg_sh.at[col_v.at[j]], add=True)

    plsc.subcore_barrier()
    pltpu.sync_copy(deg_sh.at[pl.ds(s * DPT, DPT)],
                    out_hbm.at[c, pl.ds(s * DPT, DPT)])


@functools.partial(
    pl.kernel,
    out_type=jax.ShapeDtypeStruct((NC, NPAD, H), jnp.float32),
    mesh=_MESH,
    compiler_params=pltpu.CompilerParams(use_tc_tiling_on_sc=False),
    scratch_types=[
        pltpu.VMEM((NCHK, CH), jnp.int32),
        pltpu.VMEM((NCHK, CH), jnp.int32),
        pltpu.VMEM((NCHK, CH), jnp.float32),
        pltpu.VMEM((CH, H), jnp.float32),
        pltpu.VMEM((CH, H), jnp.float32),
        pltpu.VMEM((RPT, H), jnp.float32),
        pltpu.VMEM_SHARED((NPAD, H), jnp.float32),
        pltpu.SemaphoreType.DMA,
    ],
)
def _mp_kernel(tab_hbm, row_hbm, col_hbm, ew_hbm, out_hbm,
               row_v, col_v, w_v, g_v, m_v, z_v, acc_sh, sem):
    c = lax.axis_index("c")
    s = lax.axis_index("s")
    wid = c * NS + s

    @pl.loop(0, RPT)
    def _(i):
        z_v[i, :] = jnp.zeros((H,), jnp.float32)

    pltpu.sync_copy(z_v, acc_sh.at[pl.ds(s * RPT, RPT)])
    pltpu.sync_copy(row_hbm.at[wid], row_v)
    pltpu.sync_copy(col_hbm.at[wid], col_v)
    pltpu.sync_copy(ew_hbm.at[wid], w_v)
    plsc.subcore_barrier()

    @pl.loop(0, NCHK)
    def _(j):
        pltpu.async_copy(tab_hbm.at[row_v.at[j]], g_v, sem).wait()
        for gi in range(CH // 16):
            w16 = w_v[j, pl.ds(gi * 16, 16)]
            for k in range(16):
                e = gi * 16 + k
                m_v[e, :] = g_v[e, :] * _bcast(w16, k)
        pltpu.sync_copy(m_v, acc_sh.at[col_v.at[j]], add=True)

    plsc.subcore_barrier()
    pltpu.sync_copy(acc_sh.at[pl.ds(s * RPT, RPT)],
                    out_hbm.at[c, pl.ds(s * RPT, RPT)])


def _dis_of(deg_ref):
    deg = deg_ref[0] + deg_ref[1] + 1.0
    return lax.rsqrt(deg)


def _tca_body(x_ref, w1_ref, deg_ref, t1_ref):
    dis = _dis_of(deg_ref)
    h = jnp.dot(x_ref[...], w1_ref[...], preferred_element_type=jnp.float32)
    t1_ref[...] = h * dis[:, None]


def _tcb_body(acc_ref, t1_ref, deg_ref, w2_ref, b1_ref, t2_ref):
    dis = _dis_of(deg_ref)
    pre = (acc_ref[0] + acc_ref[1] + t1_ref[...]) * dis[:, None] + b1_ref[0][None, :]
    g = jnp.maximum(pre, 0.0)
    h2 = jnp.dot(g, w2_ref[...], preferred_element_type=jnp.float32)
    t2_ref[...] = h2 * dis[:, None]


def _tcc_body(acc_ref, t2_ref, deg_ref, b2_ref, o_ref):
    dis = _dis_of(deg_ref)
    z = (acc_ref[0] + acc_ref[1] + t2_ref[...]) * dis[:, None] + b2_ref[0][None, :]
    m = jnp.max(z, axis=1, keepdims=True)
    lse = jnp.log(jnp.sum(jnp.exp(z - m), axis=1, keepdims=True)) + m
    o_ref[...] = z - lse


_TB = 1024
_TG = (N + _TB - 1) // _TB

_deg_spec = pl.BlockSpec((NC, _TB), lambda i: (0, i))
_row16_spec = pl.BlockSpec((_TB, H), lambda i: (i, 0))
_acc_spec = pl.BlockSpec((NC, _TB, H), lambda i: (0, i, 0))


def _tc_a(x, W1, deg2):
    return pl.pallas_call(
        _tca_body,
        grid=(_TG,),
        in_specs=[pl.BlockSpec((_TB, D), lambda i: (i, 0)),
                  pl.BlockSpec((D, H), lambda i: (0, 0)),
                  _deg_spec],
        out_specs=_row16_spec,
        out_shape=jax.ShapeDtypeStruct((N, H), jnp.float32),
    )(x, W1, deg2)


def _tc_b(acc1, t1, deg2, W2, b1):
    return pl.pallas_call(
        _tcb_body,
        grid=(_TG,),
        in_specs=[_acc_spec, _row16_spec, _deg_spec,
                  pl.BlockSpec((H, H), lambda i: (0, 0)),
                  pl.BlockSpec((1, H), lambda i: (0, 0))],
        out_specs=_row16_spec,
        out_shape=jax.ShapeDtypeStruct((N, H), jnp.float32),
    )(acc1, t1, deg2, W2, b1)


def _tc_c(acc2, t2, deg2, b2):
    return pl.pallas_call(
        _tcc_body,
        grid=(_TG,),
        in_specs=[_acc_spec, _row16_spec, _deg_spec,
                  pl.BlockSpec((1, H), lambda i: (0, 0))],
        out_specs=_row16_spec,
        out_shape=jax.ShapeDtypeStruct((N, H), jnp.float32),
    )(acc2, t2, deg2, b2)


def kernel(x, edge_index, edge_weight, W1, b1, W2, b2):
    row = edge_index[0].reshape(NW, NCHK, CH)
    col = edge_index[1].reshape(NW, NCHK, CH)
    ew = edge_weight.reshape(NW, NCHK, CH)

    deg2 = _deg_kernel(col, ew)
    t1 = _tc_a(x, W1, deg2)
    acc1 = _mp_kernel(t1, row, col, ew)
    t2 = _tc_b(acc1, t1, deg2, W2, b1.reshape(1, H))
    acc2 = _mp_kernel(t2, row, col, ew)
    return _tc_c(acc2, t2, deg2, b2.reshape(1, H))


# speedup vs baseline: 44.2365x; 1.6017x over previous
import functools

import jax
import jax.numpy as jnp
from jax import lax
from jax.experimental import pallas as pl
from jax.experimental.pallas import tpu as pltpu
from jax.experimental.pallas import tpu_sc as plsc

N = 10000
E = 320000
D = 128
H = 16
NC, NS = 2, 16
NW = NC * NS
EPT = E // NW
CH = 80
NCHK = EPT // CH
NPAD = 10240
DPT = NPAD // NS
RPT = NPAD // NS
DEG_LAG = 8

assert NCHK == 125

_MESH = plsc.VectorSubcoreMesh(core_axis_name="c", subcore_axis_name="s")
_SC_PARAMS = pltpu.CompilerParams(use_tc_tiling_on_sc=False)

_GDN = lax.GatherDimensionNumbers(
    offset_dims=(), collapsed_slice_dims=(0,), start_index_map=(0,))


def _bcast(v, k):
    return lax.gather(v, jnp.full((16, 1), k, jnp.int32), _GDN, (1,),
                      mode=lax.GatherScatterMode.PROMISE_IN_BOUNDS)


@functools.partial(
    pl.kernel,
    out_type=jax.ShapeDtypeStruct((NC, NPAD), jnp.float32),
    mesh=_MESH,
    compiler_params=_SC_PARAMS,
    scratch_types=[
        pltpu.VMEM((NCHK, CH), jnp.int32),
        pltpu.VMEM((NCHK, CH), jnp.float32),
        pltpu.VMEM((DPT,), jnp.float32),
        pltpu.VMEM_SHARED((NPAD,), jnp.float32),
        pltpu.SemaphoreType.DMA,
    ],
)
def _deg_kernel(col_hbm, ew_hbm, out_hbm, col_v, w_v, z_v, deg_sh, dsem):
    ci = lax.axis_index("c")
    s = lax.axis_index("s")
    wid = ci * NS + s

    @pl.loop(0, DPT // 16, unroll=8)
    def _(i):
        z_v[pl.ds(i * 16, 16)] = jnp.zeros((16,), jnp.float32)

    pltpu.sync_copy(z_v, deg_sh.at[pl.ds(s * DPT, DPT)])
    pltpu.sync_copy(col_hbm.at[wid], col_v)
    pltpu.sync_copy(ew_hbm.at[wid], w_v)
    plsc.subcore_barrier()

    def _issue(j):
        pltpu.async_copy(w_v.at[j], deg_sh.at[col_v.at[j]], dsem, add=True)

    def _wait(j):
        pltpu.make_async_copy(w_v.at[j], deg_sh.at[col_v.at[j]], dsem).wait()

    for j in range(DEG_LAG):
        _issue(j)

    @pl.loop(0, NCHK - DEG_LAG)
    def _(j):
        _wait(j)
        _issue(j + DEG_LAG)

    for t in range(DEG_LAG):
        _wait(NCHK - DEG_LAG + t)

    plsc.subcore_barrier()
    pltpu.sync_copy(deg_sh.at[pl.ds(s * DPT, DPT)],
                    out_hbm.at[ci, pl.ds(s * DPT, DPT)])


@functools.partial(
    pl.kernel,
    out_type=jax.ShapeDtypeStruct((NC, NPAD, H), jnp.float32),
    mesh=_MESH,
    compiler_params=_SC_PARAMS,
    scratch_types=[
        pltpu.VMEM((NCHK, CH), jnp.int32),
        pltpu.VMEM((NCHK, CH), jnp.int32),
        pltpu.VMEM((NCHK, CH), jnp.float32),
        pltpu.VMEM((2, CH, H), jnp.float32),
        pltpu.VMEM((2, CH, H), jnp.float32),
        pltpu.VMEM((RPT, H), jnp.float32),
        pltpu.VMEM_SHARED((NPAD, H), jnp.float32),
        pltpu.SemaphoreType.DMA((2,)),
        pltpu.SemaphoreType.DMA((2,)),
    ],
)
def _mp_kernel(tab_hbm, row_hbm, col_hbm, ew_hbm, out_hbm,
               row_v, col_v, w_v, g2, m2, z_v, acc_sh, gsem, ssem):
    ci = lax.axis_index("c")
    s = lax.axis_index("s")
    wid = ci * NS + s

    @pl.loop(0, RPT, unroll=8)
    def _(i):
        z_v[i, :] = jnp.zeros((H,), jnp.float32)

    pltpu.sync_copy(z_v, acc_sh.at[pl.ds(s * RPT, RPT)])
    pltpu.sync_copy(row_hbm.at[wid], row_v)
    pltpu.sync_copy(col_hbm.at[wid], col_v)
    pltpu.sync_copy(ew_hbm.at[wid], w_v)
    plsc.subcore_barrier()

    def g_issue(c, b):
        pltpu.async_copy(tab_hbm.at[row_v.at[c]], g2.at[b], gsem.at[b])

    def g_wait(c, b):
        pltpu.make_async_copy(tab_hbm.at[row_v.at[c]], g2.at[b],
                              gsem.at[b]).wait()

    def s_issue(c, b):
        pltpu.async_copy(m2.at[b], acc_sh.at[col_v.at[c]], ssem.at[b],
                         add=True)

    def s_wait(c, b):
        pltpu.make_async_copy(m2.at[b], acc_sh.at[col_v.at[c]],
                              ssem.at[b]).wait()

    def compute(c, b):
        for gi in range(CH // 16):
            w16 = w_v[c, pl.ds(gi * 16, 16)]
            for k in range(16):
                e = gi * 16 + k
                m2[b, e, :] = g2[b, e, :] * _bcast(w16, k)

    g_issue(0, 0)
    g_issue(1, 1)
    for b in range(2):
        g_wait(b, b)
        compute(b, b)
        s_issue(b, b)
        g_issue(b + 2, b)

    @pl.loop(1, 61)
    def _(jj):
        c0 = jj * 2
        for b in range(2):
            c = c0 + b
            g_wait(c, b)
            s_wait(c, b)
            compute(c, b)
            s_issue(c, b)
            g_issue(c + 2, b)

    g_wait(122, 0); s_wait(122, 0); compute(122, 0); s_issue(122, 0)
    g_issue(124, 0)
    g_wait(123, 1); s_wait(123, 1); compute(123, 1); s_issue(123, 1)
    g_wait(124, 0); s_wait(124, 0); compute(124, 0); s_issue(124, 0)
    s_wait(124, 0)
    s_wait(123, 1)

    plsc.subcore_barrier()
    pltpu.sync_copy(acc_sh.at[pl.ds(s * RPT, RPT)],
                    out_hbm.at[ci, pl.ds(s * RPT, RPT)])


def _dis_of(deg_ref):
    deg = deg_ref[0] + deg_ref[1] + 1.0
    return lax.rsqrt(deg)


def _tca_body(x_ref, w1_ref, deg_ref, t1_ref):
    dis = _dis_of(deg_ref)
    h = jnp.dot(x_ref[...], w1_ref[...], preferred_element_type=jnp.float32)
    t1_ref[...] = h * dis[:, None]


def _tcb_body(acc_ref, t1_ref, deg_ref, w2_ref, b1_ref, t2_ref):
    dis = _dis_of(deg_ref)
    pre = (acc_ref[0] + acc_ref[1] + t1_ref[...]) * dis[:, None] + b1_ref[0][None, :]
    g = jnp.maximum(pre, 0.0)
    h2 = jnp.dot(g, w2_ref[...], preferred_element_type=jnp.float32)
    t2_ref[...] = h2 * dis[:, None]


def _tcc_body(acc_ref, t2_ref, deg_ref, b2_ref, o_ref):
    dis = _dis_of(deg_ref)
    z = (acc_ref[0] + acc_ref[1] + t2_ref[...]) * dis[:, None] + b2_ref[0][None, :]
    m = jnp.max(z, axis=1, keepdims=True)
    lse = jnp.log(jnp.sum(jnp.exp(z - m), axis=1, keepdims=True)) + m
    o_ref[...] = z - lse


_TB = 1024
_TG = (N + _TB - 1) // _TB

_deg_spec = pl.BlockSpec((NC, _TB), lambda i: (0, i))
_row16_spec = pl.BlockSpec((_TB, H), lambda i: (i, 0))
_acc_spec = pl.BlockSpec((NC, _TB, H), lambda i: (0, i, 0))


def _tc_a(x, W1, deg2):
    return pl.pallas_call(
        _tca_body,
        grid=(_TG,),
        in_specs=[pl.BlockSpec((_TB, D), lambda i: (i, 0)),
                  pl.BlockSpec((D, H), lambda i: (0, 0)),
                  _deg_spec],
        out_specs=_row16_spec,
        out_shape=jax.ShapeDtypeStruct((N, H), jnp.float32),
    )(x, W1, deg2)


def _tc_b(acc1, t1, deg2, W2, b1):
    return pl.pallas_call(
        _tcb_body,
        grid=(_TG,),
        in_specs=[_acc_spec, _row16_spec, _deg_spec,
                  pl.BlockSpec((H, H), lambda i: (0, 0)),
                  pl.BlockSpec((1, H), lambda i: (0, 0))],
        out_specs=_row16_spec,
        out_shape=jax.ShapeDtypeStruct((N, H), jnp.float32),
    )(acc1, t1, deg2, W2, b1)


def _tc_c(acc2, t2, deg2, b2):
    return pl.pallas_call(
        _tcc_body,
        grid=(_TG,),
        in_specs=[_acc_spec, _row16_spec, _deg_spec,
                  pl.BlockSpec((1, H), lambda i: (0, 0))],
        out_specs=_row16_spec,
        out_shape=jax.ShapeDtypeStruct((N, H), jnp.float32),
    )(acc2, t2, deg2, b2)


def kernel(x, edge_index, edge_weight, W1, b1, W2, b2):
    row = edge_index[0].reshape(NW, NCHK, CH)
    col = edge_index[1].reshape(NW, NCHK, CH)
    ew = edge_weight.reshape(NW, NCHK, CH)

    deg2 = _deg_kernel(col, ew)
    t1 = _tc_a(x, W1, deg2)
    acc1 = _mp_kernel(t1, row, col, ew)
    t2 = _tc_b(acc1, t1, deg2, W2, b1.reshape(1, H))
    acc2 = _mp_kernel(t2, row, col, ew)
    return _tc_c(acc2, t2, deg2, b2.reshape(1, H))


# speedup vs baseline: 57.6976x; 1.3043x over previous
import functools

import jax
import jax.numpy as jnp
from jax import lax
from jax.experimental import pallas as pl
from jax.experimental.pallas import tpu as pltpu
from jax.experimental.pallas import tpu_sc as plsc

N = 10000
E = 320000
D = 128
H = 16
NC, NS = 2, 16
NW = NC * NS
EPT = E // NW
CH = 80
NCHK = EPT // CH
NPAD = 10240
DPT = NPAD // NS
RPT = NPAD // NS
DEG_LAG = 8
NBUF = 5

assert NCHK == 125 and NCHK % NBUF == 0

_MESH = plsc.VectorSubcoreMesh(core_axis_name="c", subcore_axis_name="s")
_SC_PARAMS = pltpu.CompilerParams(use_tc_tiling_on_sc=False)

_GDN = lax.GatherDimensionNumbers(
    offset_dims=(), collapsed_slice_dims=(0,), start_index_map=(0,))


def _bcast(v, k):
    return lax.gather(v, jnp.full((16, 1), k, jnp.int32), _GDN, (1,),
                      mode=lax.GatherScatterMode.PROMISE_IN_BOUNDS)


@functools.partial(
    pl.kernel,
    out_type=jax.ShapeDtypeStruct((NC, NPAD), jnp.float32),
    mesh=_MESH,
    compiler_params=_SC_PARAMS,
    scratch_types=[
        pltpu.VMEM((NCHK, CH), jnp.int32),
        pltpu.VMEM((NCHK, CH), jnp.float32),
        pltpu.VMEM((DPT,), jnp.float32),
        pltpu.VMEM_SHARED((NPAD,), jnp.float32),
        pltpu.SemaphoreType.DMA,
    ],
)
def _deg_kernel(col_hbm, ew_hbm, out_hbm, col_v, w_v, z_v, deg_sh, dsem):
    ci = lax.axis_index("c")
    s = lax.axis_index("s")
    wid = ci * NS + s

    @pl.loop(0, DPT // 16, unroll=8)
    def _(i):
        z_v[pl.ds(i * 16, 16)] = jnp.zeros((16,), jnp.float32)

    pltpu.sync_copy(z_v, deg_sh.at[pl.ds(s * DPT, DPT)])
    pltpu.sync_copy(col_hbm.at[wid], col_v)
    pltpu.sync_copy(ew_hbm.at[wid], w_v)
    plsc.subcore_barrier()

    def _issue(j):
        pltpu.async_copy(w_v.at[j], deg_sh.at[col_v.at[j]], dsem, add=True)

    def _wait(j):
        pltpu.make_async_copy(w_v.at[j], deg_sh.at[col_v.at[j]], dsem).wait()

    for j in range(DEG_LAG):
        _issue(j)

    @pl.loop(0, NCHK - DEG_LAG)
    def _(j):
        _wait(j)
        _issue(j + DEG_LAG)

    for t in range(DEG_LAG):
        _wait(NCHK - DEG_LAG + t)

    plsc.subcore_barrier()
    pltpu.sync_copy(deg_sh.at[pl.ds(s * DPT, DPT)],
                    out_hbm.at[ci, pl.ds(s * DPT, DPT)])


@functools.partial(
    pl.kernel,
    out_type=jax.ShapeDtypeStruct((NC, NPAD, H), jnp.float32),
    mesh=_MESH,
    compiler_params=_SC_PARAMS,
    scratch_types=[
        pltpu.VMEM((NCHK, CH), jnp.int32),
        pltpu.VMEM((NCHK, CH), jnp.int32),
        pltpu.VMEM((NCHK, CH), jnp.float32),
        pltpu.VMEM((NBUF, CH, H), jnp.float32),
        pltpu.VMEM((NBUF, CH, H), jnp.float32),
        pltpu.VMEM((RPT, H), jnp.float32),
        pltpu.VMEM_SHARED((NPAD, H), jnp.float32),
        pltpu.SemaphoreType.DMA((NBUF,)),
        pltpu.SemaphoreType.DMA((NBUF,)),
    ],
)
def _mp_kernel(tab_hbm, row_hbm, col_hbm, ew_hbm, out_hbm,
               row_v, col_v, w_v, g2, m2, z_v, acc_sh, gsem, ssem):
    ci = lax.axis_index("c")
    s = lax.axis_index("s")
    wid = ci * NS + s

    @pl.loop(0, RPT, unroll=8)
    def _(i):
        z_v[i, :] = jnp.zeros((H,), jnp.float32)

    pltpu.sync_copy(z_v, acc_sh.at[pl.ds(s * RPT, RPT)])
    pltpu.sync_copy(row_hbm.at[wid], row_v)
    pltpu.sync_copy(col_hbm.at[wid], col_v)
    pltpu.sync_copy(ew_hbm.at[wid], w_v)
    plsc.subcore_barrier()

    def g_issue(c, b):
        pltpu.async_copy(tab_hbm.at[row_v.at[c]], g2.at[b], gsem.at[b])

    def g_wait(c, b):
        pltpu.make_async_copy(tab_hbm.at[row_v.at[c]], g2.at[b],
                              gsem.at[b]).wait()

    def s_issue(c, b):
        pltpu.async_copy(m2.at[b], acc_sh.at[col_v.at[c]], ssem.at[b],
                         add=True)

    def s_wait(c, b):
        pltpu.make_async_copy(m2.at[b], acc_sh.at[col_v.at[c]],
                              ssem.at[b]).wait()

    def compute(c, b):
        for gi in range(CH // 16):
            w16 = w_v[c, pl.ds(gi * 16, 16)]
            for k in range(16):
                e = gi * 16 + k
                m2[b, e, :] = g2[b, e, :] * _bcast(w16, k)

    for b in range(NBUF):
        g_issue(b, b)
    for b in range(NBUF):
        g_wait(b, b)
        compute(b, b)
        s_issue(b, b)
        g_issue(b + NBUF, b)

    @pl.loop(1, NCHK // NBUF - 1)
    def _(jj):
        c0 = jj * NBUF
        for b in range(NBUF):
            c = c0 + b
            g_wait(c, b)
            s_wait(c, b)
            compute(c, b)
            s_issue(c, b)
            g_issue(c + NBUF, b)

    for b in range(NBUF):
        c = NCHK - NBUF + b
        g_wait(c, b)
        s_wait(c, b)
        compute(c, b)
        s_issue(c, b)
    for b in range(NBUF):
        s_wait(NCHK - NBUF + b, b)

    plsc.subcore_barrier()
    pltpu.sync_copy(acc_sh.at[pl.ds(s * RPT, RPT)],
                    out_hbm.at[ci, pl.ds(s * RPT, RPT)])


def _dis_of(deg_ref):
    deg = deg_ref[0] + deg_ref[1] + 1.0
    return lax.rsqrt(deg)


def _tca_body(x_ref, w1_ref, deg_ref, t1_ref):
    dis = _dis_of(deg_ref)
    h = jnp.dot(x_ref[...], w1_ref[...], preferred_element_type=jnp.float32)
    t1_ref[...] = h * dis[:, None]


def _tcb_body(acc_ref, t1_ref, deg_ref, w2_ref, b1_ref, t2_ref):
    dis = _dis_of(deg_ref)
    pre = (acc_ref[0] + acc_ref[1] + t1_ref[...]) * dis[:, None] + b1_ref[0][None, :]
    g = jnp.maximum(pre, 0.0)
    h2 = jnp.dot(g, w2_ref[...], preferred_element_type=jnp.float32)
    t2_ref[...] = h2 * dis[:, None]


def _tcc_body(acc_ref, t2_ref, deg_ref, b2_ref, o_ref):
    dis = _dis_of(deg_ref)
    z = (acc_ref[0] + acc_ref[1] + t2_ref[...]) * dis[:, None] + b2_ref[0][None, :]
    m = jnp.max(z, axis=1, keepdims=True)
    lse = jnp.log(jnp.sum(jnp.exp(z - m), axis=1, keepdims=True)) + m
    o_ref[...] = z - lse


_TB = 1024
_TG = (N + _TB - 1) // _TB

_deg_spec = pl.BlockSpec((NC, _TB), lambda i: (0, i))
_row16_spec = pl.BlockSpec((_TB, H), lambda i: (i, 0))
_acc_spec = pl.BlockSpec((NC, _TB, H), lambda i: (0, i, 0))


def _tc_a(x, W1, deg2):
    return pl.pallas_call(
        _tca_body,
        grid=(_TG,),
        in_specs=[pl.BlockSpec((_TB, D), lambda i: (i, 0)),
                  pl.BlockSpec((D, H), lambda i: (0, 0)),
                  _deg_spec],
        out_specs=_row16_spec,
        out_shape=jax.ShapeDtypeStruct((N, H), jnp.float32),
    )(x, W1, deg2)


def _tc_b(acc1, t1, deg2, W2, b1):
    return pl.pallas_call(
        _tcb_body,
        grid=(_TG,),
        in_specs=[_acc_spec, _row16_spec, _deg_spec,
                  pl.BlockSpec((H, H), lambda i: (0, 0)),
                  pl.BlockSpec((1, H), lambda i: (0, 0))],
        out_specs=_row16_spec,
        out_shape=jax.ShapeDtypeStruct((N, H), jnp.float32),
    )(acc1, t1, deg2, W2, b1)


def _tc_c(acc2, t2, deg2, b2):
    return pl.pallas_call(
        _tcc_body,
        grid=(_TG,),
        in_specs=[_acc_spec, _row16_spec, _deg_spec,
                  pl.BlockSpec((1, H), lambda i: (0, 0))],
        out_specs=_row16_spec,
        out_shape=jax.ShapeDtypeStruct((N, H), jnp.float32),
    )(acc2, t2, deg2, b2)


def kernel(x, edge_index, edge_weight, W1, b1, W2, b2):
    row = edge_index[0].reshape(NW, NCHK, CH)
    col = edge_index[1].reshape(NW, NCHK, CH)
    ew = edge_weight.reshape(NW, NCHK, CH)

    deg2 = _deg_kernel(col, ew)
    t1 = _tc_a(x, W1, deg2)
    acc1 = _mp_kernel(t1, row, col, ew)
    t2 = _tc_b(acc1, t1, deg2, W2, b1.reshape(1, H))
    acc2 = _mp_kernel(t2, row, col, ew)
    return _tc_c(acc2, t2, deg2, b2.reshape(1, H))


# speedup vs baseline: 62.7155x; 1.0870x over previous
import functools

import jax
import jax.numpy as jnp
from jax import lax
from jax.experimental import pallas as pl
from jax.experimental.pallas import tpu as pltpu
from jax.experimental.pallas import tpu_sc as plsc

N = 10000
E = 320000
D = 128
H = 16
NC, NS = 2, 16
NW = NC * NS
EPT = E // NW
CH = 80
NCHK = EPT // CH
NPAD = 10240
DPT = NPAD // NS
RPT = NPAD // NS
DEG_LAG = 8
NBUF = 5

assert NCHK == 125 and NCHK % NBUF == 0

_MESH = plsc.VectorSubcoreMesh(core_axis_name="c", subcore_axis_name="s")
_SC_PARAMS = pltpu.CompilerParams(use_tc_tiling_on_sc=False)

_GDN = lax.GatherDimensionNumbers(
    offset_dims=(), collapsed_slice_dims=(0,), start_index_map=(0,))


def _bcast(v, k):
    return lax.gather(v, jnp.full((16, 1), k, jnp.int32), _GDN, (1,),
                      mode=lax.GatherScatterMode.PROMISE_IN_BOUNDS)


@functools.partial(
    pl.kernel,
    out_type=jax.ShapeDtypeStruct((NC, NPAD), jnp.float32),
    mesh=_MESH,
    compiler_params=_SC_PARAMS,
    scratch_types=[
        pltpu.VMEM((NCHK, CH), jnp.int32),
        pltpu.VMEM((NCHK, CH), jnp.float32),
        pltpu.VMEM((DPT,), jnp.float32),
        pltpu.VMEM_SHARED((NPAD,), jnp.float32),
        pltpu.SemaphoreType.DMA,
    ],
)
def _deg_kernel(col_hbm, ew_hbm, out_hbm, col_v, w_v, z_v, deg_sh, dsem):
    ci = lax.axis_index("c")
    s = lax.axis_index("s")
    wid = ci * NS + s

    @pl.loop(0, DPT // 16, unroll=8)
    def _(i):
        z_v[pl.ds(i * 16, 16)] = jnp.zeros((16,), jnp.float32)

    pltpu.sync_copy(z_v, deg_sh.at[pl.ds(s * DPT, DPT)])
    pltpu.sync_copy(col_hbm.at[wid], col_v)
    pltpu.sync_copy(ew_hbm.at[wid], w_v)
    plsc.subcore_barrier()

    def _issue(j):
        pltpu.async_copy(w_v.at[j], deg_sh.at[col_v.at[j]], dsem, add=True)

    def _wait(j):
        pltpu.make_async_copy(w_v.at[j], deg_sh.at[col_v.at[j]], dsem).wait()

    for j in range(DEG_LAG):
        _issue(j)

    @pl.loop(0, NCHK - DEG_LAG)
    def _(j):
        _wait(j)
        _issue(j + DEG_LAG)

    for t in range(DEG_LAG):
        _wait(NCHK - DEG_LAG + t)

    plsc.subcore_barrier()
    pltpu.sync_copy(deg_sh.at[pl.ds(s * DPT, DPT)],
                    out_hbm.at[ci, pl.ds(s * DPT, DPT)])


@functools.partial(
    pl.kernel,
    out_type=jax.ShapeDtypeStruct((NC, NPAD, H), jnp.float32),
    mesh=_MESH,
    compiler_params=_SC_PARAMS,
    scratch_types=[
        pltpu.VMEM((NCHK, CH), jnp.int32),
        pltpu.VMEM((NCHK, CH), jnp.int32),
        pltpu.VMEM((NCHK, CH), jnp.float32),
        pltpu.VMEM((NBUF, CH, H), jnp.float32),
        pltpu.VMEM((NBUF, CH, H), jnp.float32),
        pltpu.VMEM((RPT, H), jnp.float32),
        pltpu.VMEM_SHARED((NPAD, H), jnp.float32),
        pltpu.VMEM_SHARED((NPAD, H), jnp.float32),
        pltpu.SemaphoreType.DMA((NBUF,)),
        pltpu.SemaphoreType.DMA((NBUF,)),
    ],
)
def _mp_kernel(tab_hbm, row_hbm, col_hbm, ew_hbm, out_hbm,
               row_v, col_v, w_v, g2, m2, z_v, acc_sh, tab_sh, gsem, ssem):
    ci = lax.axis_index("c")
    s = lax.axis_index("s")
    wid = ci * NS + s

    @pl.loop(0, RPT, unroll=8)
    def _(i):
        z_v[i, :] = jnp.zeros((H,), jnp.float32)

    pltpu.sync_copy(z_v, acc_sh.at[pl.ds(s * RPT, RPT)])
    pltpu.sync_copy(tab_hbm.at[pl.ds(s * RPT, RPT)],
                    tab_sh.at[pl.ds(s * RPT, RPT)])
    pltpu.sync_copy(row_hbm.at[wid], row_v)
    pltpu.sync_copy(col_hbm.at[wid], col_v)
    pltpu.sync_copy(ew_hbm.at[wid], w_v)
    plsc.subcore_barrier()

    def g_issue(c, b):
        pltpu.async_copy(tab_sh.at[row_v.at[c]], g2.at[b], gsem.at[b])

    def g_wait(c, b):
        pltpu.make_async_copy(tab_sh.at[row_v.at[c]], g2.at[b],
                              gsem.at[b]).wait()

    def s_issue(c, b):
        pltpu.async_copy(m2.at[b], acc_sh.at[col_v.at[c]], ssem.at[b],
                         add=True)

    def s_wait(c, b):
        pltpu.make_async_copy(m2.at[b], acc_sh.at[col_v.at[c]],
                              ssem.at[b]).wait()

    def compute(c, b):
        for gi in range(CH // 16):
            w16 = w_v[c, pl.ds(gi * 16, 16)]
            for k in range(16):
                e = gi * 16 + k
                m2[b, e, :] = g2[b, e, :] * _bcast(w16, k)

    for b in range(NBUF):
        g_issue(b, b)
    for b in range(NBUF):
        g_wait(b, b)
        compute(b, b)
        s_issue(b, b)
        g_issue(b + NBUF, b)

    @pl.loop(1, NCHK // NBUF - 1)
    def _(jj):
        c0 = jj * NBUF
        for b in range(NBUF):
            c = c0 + b
            g_wait(c, b)
            s_wait(c, b)
            compute(c, b)
            s_issue(c, b)
            g_issue(c + NBUF, b)

    for b in range(NBUF):
        c = NCHK - NBUF + b
        g_wait(c, b)
        s_wait(c, b)
        compute(c, b)
        s_issue(c, b)
    for b in range(NBUF):
        s_wait(NCHK - NBUF + b, b)

    plsc.subcore_barrier()
    pltpu.sync_copy(acc_sh.at[pl.ds(s * RPT, RPT)],
                    out_hbm.at[ci, pl.ds(s * RPT, RPT)])


def _dis_of(deg_ref):
    deg = deg_ref[0] + deg_ref[1] + 1.0
    return lax.rsqrt(deg)


def _tca_body(x_ref, w1_ref, deg_ref, t1_ref):
    dis = _dis_of(deg_ref)
    h = jnp.dot(x_ref[...], w1_ref[...], preferred_element_type=jnp.float32)
    t1_ref[...] = h * dis[:, None]


def _tcb_body(acc_ref, t1_ref, deg_ref, w2_ref, b1_ref, t2_ref):
    dis = _dis_of(deg_ref)
    pre = (acc_ref[0] + acc_ref[1] + t1_ref[...]) * dis[:, None] + b1_ref[0][None, :]
    g = jnp.maximum(pre, 0.0)
    h2 = jnp.dot(g, w2_ref[...], preferred_element_type=jnp.float32)
    t2_ref[...] = h2 * dis[:, None]


def _tcc_body(acc_ref, t2_ref, deg_ref, b2_ref, o_ref):
    dis = _dis_of(deg_ref)
    z = (acc_ref[0] + acc_ref[1] + t2_ref[...]) * dis[:, None] + b2_ref[0][None, :]
    m = jnp.max(z, axis=1, keepdims=True)
    lse = jnp.log(jnp.sum(jnp.exp(z - m), axis=1, keepdims=True)) + m
    o_ref[...] = z - lse


_TB = 1024
_TG = (N + _TB - 1) // _TB

_deg_spec = pl.BlockSpec((NC, _TB), lambda i: (0, i))
_row16_spec = pl.BlockSpec((_TB, H), lambda i: (i, 0))
_acc_spec = pl.BlockSpec((NC, _TB, H), lambda i: (0, i, 0))


def _tc_a(x, W1, deg2):
    return pl.pallas_call(
        _tca_body,
        grid=(_TG,),
        in_specs=[pl.BlockSpec((_TB, D), lambda i: (i, 0)),
                  pl.BlockSpec((D, H), lambda i: (0, 0)),
                  _deg_spec],
        out_specs=_row16_spec,
        out_shape=jax.ShapeDtypeStruct((NPAD, H), jnp.float32),
    )(x, W1, deg2)


def _tc_b(acc1, t1, deg2, W2, b1):
    return pl.pallas_call(
        _tcb_body,
        grid=(_TG,),
        in_specs=[_acc_spec, _row16_spec, _deg_spec,
                  pl.BlockSpec((H, H), lambda i: (0, 0)),
                  pl.BlockSpec((1, H), lambda i: (0, 0))],
        out_specs=_row16_spec,
        out_shape=jax.ShapeDtypeStruct((NPAD, H), jnp.float32),
    )(acc1, t1, deg2, W2, b1)


def _tc_c(acc2, t2, deg2, b2):
    return pl.pallas_call(
        _tcc_body,
        grid=(_TG,),
        in_specs=[_acc_spec, _row16_spec, _deg_spec,
                  pl.BlockSpec((1, H), lambda i: (0, 0))],
        out_specs=_row16_spec,
        out_shape=jax.ShapeDtypeStruct((N, H), jnp.float32),
    )(acc2, t2, deg2, b2)


def kernel(x, edge_index, edge_weight, W1, b1, W2, b2):
    row = edge_index[0].reshape(NW, NCHK, CH)
    col = edge_index[1].reshape(NW, NCHK, CH)
    ew = edge_weight.reshape(NW, NCHK, CH)

    deg2 = _deg_kernel(col, ew)
    t1 = _tc_a(x, W1, deg2)
    acc1 = _mp_kernel(t1, row, col, ew)
    t2 = _tc_b(acc1, t1, deg2, W2, b1.reshape(1, H))
    acc2 = _mp_kernel(t2, row, col, ew)
    return _tc_c(acc2, t2, deg2, b2.reshape(1, H))


# speedup vs baseline: 65.5834x; 1.0457x over previous
import functools

import jax
import jax.numpy as jnp
from jax import lax
from jax.experimental import pallas as pl
from jax.experimental.pallas import tpu as pltpu
from jax.experimental.pallas import tpu_sc as plsc

N = 10000
E = 320000
D = 128
H = 16
NC, NS = 2, 16
NW = NC * NS
EPT = E // NW
CH = 80
NCHK = EPT // CH
NPAD = 10240
DPT = NPAD // NS
RPT = NPAD // NS
DEG_LAG = 8
NBUF = 5

assert NCHK == 125 and NCHK % NBUF == 0

_MESH = plsc.VectorSubcoreMesh(core_axis_name="c", subcore_axis_name="s")
_SC_PARAMS = pltpu.CompilerParams(use_tc_tiling_on_sc=False)

_GDN = lax.GatherDimensionNumbers(
    offset_dims=(), collapsed_slice_dims=(0,), start_index_map=(0,))


def _bcast(v, k):
    return lax.gather(v, jnp.full((16, 1), k, jnp.int32), _GDN, (1,),
                      mode=lax.GatherScatterMode.PROMISE_IN_BOUNDS)


@functools.partial(
    pl.kernel,
    out_type=jax.ShapeDtypeStruct((NC, NPAD), jnp.float32),
    mesh=_MESH,
    compiler_params=_SC_PARAMS,
    scratch_types=[
        pltpu.VMEM((NCHK, CH), jnp.int32),
        pltpu.VMEM((NCHK, CH), jnp.float32),
        pltpu.VMEM((DPT,), jnp.float32),
        pltpu.VMEM_SHARED((NPAD,), jnp.float32),
        pltpu.SemaphoreType.DMA,
    ],
)
def _deg_kernel(col_hbm, ew_hbm, out_hbm, col_v, w_v, z_v, deg_sh, dsem):
    ci = lax.axis_index("c")
    s = lax.axis_index("s")
    wid = ci * NS + s

    @pl.loop(0, DPT // 16, unroll=8)
    def _(i):
        z_v[pl.ds(i * 16, 16)] = jnp.zeros((16,), jnp.float32)

    pltpu.sync_copy(z_v, deg_sh.at[pl.ds(s * DPT, DPT)])
    pltpu.sync_copy(col_hbm.at[wid], col_v)
    pltpu.sync_copy(ew_hbm.at[wid], w_v)
    plsc.subcore_barrier()

    def _issue(j):
        pltpu.async_copy(w_v.at[j], deg_sh.at[col_v.at[j]], dsem, add=True)

    def _wait(j):
        pltpu.make_async_copy(w_v.at[j], deg_sh.at[col_v.at[j]], dsem).wait()

    for j in range(DEG_LAG):
        _issue(j)

    @pl.loop(0, NCHK - DEG_LAG)
    def _(j):
        _wait(j)
        _issue(j + DEG_LAG)

    for t in range(DEG_LAG):
        _wait(NCHK - DEG_LAG + t)

    plsc.subcore_barrier()
    pltpu.sync_copy(deg_sh.at[pl.ds(s * DPT, DPT)],
                    out_hbm.at[ci, pl.ds(s * DPT, DPT)])


@functools.partial(
    pl.kernel,
    out_type=jax.ShapeDtypeStruct((NC, NPAD, H), jnp.float32),
    mesh=_MESH,
    compiler_params=_SC_PARAMS,
    scratch_types=[
        pltpu.VMEM((NCHK, CH), jnp.int32),
        pltpu.VMEM((NCHK, CH), jnp.int32),
        pltpu.VMEM((NCHK, CH), jnp.float32),
        pltpu.VMEM((NBUF, CH, H), jnp.float32),
        pltpu.VMEM((NBUF, CH, H), jnp.float32),
        pltpu.VMEM((RPT, H), jnp.float32),
        pltpu.VMEM_SHARED((NPAD, H), jnp.float32),
        pltpu.VMEM_SHARED((NPAD, H), jnp.float32),
        pltpu.SemaphoreType.DMA((NBUF,)),
        pltpu.SemaphoreType.DMA((NBUF,)),
    ],
)
def _mp_kernel(tab_hbm, row_hbm, col_hbm, ew_hbm, out_hbm,
               row_v, col_v, w_v, g2, m2, z_v, acc_sh, tab_sh, gsem, ssem):
    ci = lax.axis_index("c")
    s = lax.axis_index("s")
    wid = ci * NS + s

    @pl.loop(0, RPT, unroll=8)
    def _(i):
        z_v[i, :] = jnp.zeros((H,), jnp.float32)

    pltpu.sync_copy(z_v, acc_sh.at[pl.ds(s * RPT, RPT)])
    pltpu.sync_copy(tab_hbm.at[pl.ds(s * RPT, RPT)],
                    tab_sh.at[pl.ds(s * RPT, RPT)])
    pltpu.sync_copy(row_hbm.at[wid], row_v)
    pltpu.sync_copy(col_hbm.at[wid], col_v)
    pltpu.sync_copy(ew_hbm.at[wid], w_v)
    plsc.subcore_barrier()

    def g_issue(c, b):
        pltpu.async_copy(tab_sh.at[row_v.at[c]], g2.at[b], gsem.at[b])

    def g_wait(c, b):
        pltpu.make_async_copy(tab_sh.at[row_v.at[c]], g2.at[b],
                              gsem.at[b]).wait()

    def s_issue(c, b):
        pltpu.async_copy(m2.at[b], acc_sh.at[col_v.at[c]], ssem.at[b],
                         add=True)

    def s_wait(c, b):
        pltpu.make_async_copy(m2.at[b], acc_sh.at[col_v.at[c]],
                              ssem.at[b]).wait()

    def compute(c, b):
        for gi in range(CH // 16):
            w16 = w_v[c, pl.ds(gi * 16, 16)]
            for k in range(16):
                e = gi * 16 + k
                m2[b, e, :] = g2[b, e, :] * _bcast(w16, k)

    for b in range(NBUF):
        g_issue(b, b)
    for b in range(NBUF):
        g_wait(b, b)
        compute(b, b)
        s_issue(b, b)
        g_issue(b + NBUF, b)

    @pl.loop(1, NCHK // NBUF - 1)
    def _(jj):
        c0 = jj * NBUF
        for b in range(NBUF):
            c = c0 + b
            g_wait(c, b)
            s_wait(c, b)
            compute(c, b)
            s_issue(c, b)
            g_issue(c + NBUF, b)

    for b in range(NBUF):
        c = NCHK - NBUF + b
        g_wait(c, b)
        s_wait(c, b)
        compute(c, b)
        s_issue(c, b)
    for b in range(NBUF):
        s_wait(NCHK - NBUF + b, b)

    plsc.subcore_barrier()
    pltpu.sync_copy(acc_sh.at[pl.ds(s * RPT, RPT)],
                    out_hbm.at[ci, pl.ds(s * RPT, RPT)])


def _dis_of(deg_ref):
    deg = deg_ref[0] + deg_ref[1] + 1.0
    return lax.rsqrt(deg)


def _tca_body(x_ref, w1_ref, deg_ref, t1_ref):
    dis = _dis_of(deg_ref)
    h = jnp.dot(x_ref[...], w1_ref[...], preferred_element_type=jnp.float32)
    t1_ref[0:N, :] = h * dis[0:N, None]
    t1_ref[N:NPAD, :] = jnp.zeros((NPAD - N, H), jnp.float32)


def _tcb_body(acc_ref, t1_ref, deg_ref, w2_ref, b1_ref, t2_ref):
    dis = _dis_of(deg_ref)
    pre = (acc_ref[0] + acc_ref[1] + t1_ref[...]) * dis[:, None] + b1_ref[0][None, :]
    g = jnp.maximum(pre, 0.0)
    h2 = jnp.dot(g, w2_ref[...], preferred_element_type=jnp.float32)
    t2_ref[...] = h2 * dis[:, None]


def _tcc_body(acc_ref, t2_ref, deg_ref, b2_ref, o_ref):
    dis = _dis_of(deg_ref)
    z = (acc_ref[0] + acc_ref[1] + t2_ref[...]) * dis[:, None] + b2_ref[0][None, :]
    m = jnp.max(z, axis=1, keepdims=True)
    lse = jnp.log(jnp.sum(jnp.exp(z - m), axis=1, keepdims=True)) + m
    o_ref[...] = (z - lse)[0:N, :]


def _tc_a(x, W1, deg2):
    return pl.pallas_call(
        _tca_body,
        out_shape=jax.ShapeDtypeStruct((NPAD, H), jnp.float32),
    )(x, W1, deg2)


def _tc_b(acc1, t1, deg2, W2, b1):
    return pl.pallas_call(
        _tcb_body,
        out_shape=jax.ShapeDtypeStruct((NPAD, H), jnp.float32),
    )(acc1, t1, deg2, W2, b1)


def _tc_c(acc2, t2, deg2, b2):
    return pl.pallas_call(
        _tcc_body,
        out_shape=jax.ShapeDtypeStruct((N, H), jnp.float32),
    )(acc2, t2, deg2, b2)


def kernel(x, edge_index, edge_weight, W1, b1, W2, b2):
    row = edge_index[0].reshape(NW, NCHK, CH)
    col = edge_index[1].reshape(NW, NCHK, CH)
    ew = edge_weight.reshape(NW, NCHK, CH)

    deg2 = _deg_kernel(col, ew)
    t1 = _tc_a(x, W1, deg2)
    acc1 = _mp_kernel(t1, row, col, ew)
    t2 = _tc_b(acc1, t1, deg2, W2, b1.reshape(1, H))
    acc2 = _mp_kernel(t2, row, col, ew)
    return _tc_c(acc2, t2, deg2, b2.reshape(1, H))
